# Initial kernel scaffold; baseline (speedup 1.0000x reference)
import jax
import jax.numpy as jnp
from jax.experimental import pallas as pl


def kernel(atom_feats, atom_sizes, bond_feats, bond_sizes, global_feats, w_atom, w_bond):
    raise NotImplementedError("write your pallas kernel here")


# speedup vs baseline: 7.0008x; 7.0008x over previous
import jax
import jax.numpy as jnp
from jax.experimental import pallas as pl
from jax.experimental.pallas import tpu as pltpu

_BLK = 256


def _body(scal, af_hbm, bf_hbm, gf_blk, wa, wb, out_blk, va, vb, sems):
    b = pl.program_id(0)
    nseg = pl.num_programs(0)
    slot = jax.lax.rem(b, 2)

    def issue(seg, slot):
        pltpu.make_async_copy(
            af_hbm.at[pl.ds(scal[0, seg], _BLK)], va.at[slot], sems.at[0, slot]
        ).start()
        pltpu.make_async_copy(
            bf_hbm.at[pl.ds(scal[3, seg], _BLK)], vb.at[slot], sems.at[1, slot]
        ).start()

    @pl.when(b == 0)
    def _():
        issue(0, 0)

    @pl.when(b + 1 < nseg)
    def _():
        issue(b + 1, 1 - slot)

    pltpu.make_async_copy(
        af_hbm.at[pl.ds(scal[0, b], _BLK)], va.at[slot], sems.at[0, slot]
    ).wait()
    pltpu.make_async_copy(
        bf_hbm.at[pl.ds(scal[3, b], _BLK)], vb.at[slot], sems.at[1, slot]
    ).wait()

    def branch(x, shift, size, w_row):
        score = jnp.sum(x * w_row, axis=1, keepdims=True)
        score = jnp.where(score >= 0.0, score, 0.2 * score)
        local = jax.lax.broadcasted_iota(jnp.int32, (_BLK, 1), 0) - shift
        valid = (local >= 0) & (local < size)
        m = jnp.max(jnp.where(valid, score, -jnp.inf))
        m = jnp.where(jnp.isfinite(m), m, 0.0)
        e = jnp.where(valid, jnp.exp(score - m), 0.0)
        s = jnp.sum(e)
        alpha = e / jnp.where(s > 0.0, s, 1.0)
        return jnp.sum(x * alpha, axis=0, keepdims=True)

    ra = branch(va[slot], scal[1, b], scal[2, b], wa[...])
    rb = branch(vb[slot], scal[4, b], scal[5, b], wb[...])
    out_blk[0, :, 0:128] = ra
    out_blk[0, :, 128:256] = rb
    out_blk[0, :, 256:384] = gf_blk[0]


def kernel(atom_feats, atom_sizes, bond_feats, bond_sizes, global_feats, w_atom, w_bond):
    N, D = atom_feats.shape
    B = global_feats.shape[0]

    def mk(sizes):
        sizes = sizes.astype(jnp.int32)
        cs = jnp.cumsum(sizes)
        starts = jnp.concatenate([jnp.zeros((1,), jnp.int32), cs[:-1]])
        sc = jnp.minimum(starts, N - _BLK)
        return sc, starts - sc, sizes

    sca, sha, sza = mk(atom_sizes)
    scb, shb, szb = mk(bond_sizes)
    scal = jnp.stack([sca, sha, sza, scb, shb, szb])

    grid_spec = pltpu.PrefetchScalarGridSpec(
        num_scalar_prefetch=1,
        grid=(B,),
        in_specs=[
            pl.BlockSpec(memory_space=pltpu.MemorySpace.HBM),
            pl.BlockSpec(memory_space=pltpu.MemorySpace.HBM),
            pl.BlockSpec((1, 1, D), lambda b, s: (b, 0, 0)),
            pl.BlockSpec((1, D), lambda b, s: (0, 0)),
            pl.BlockSpec((1, D), lambda b, s: (0, 0)),
        ],
        out_specs=pl.BlockSpec((1, 1, 3 * D), lambda b, s: (b, 0, 0)),
        scratch_shapes=[
            pltpu.VMEM((2, _BLK, D), jnp.float32),
            pltpu.VMEM((2, _BLK, D), jnp.float32),
            pltpu.SemaphoreType.DMA((2, 2)),
        ],
    )
    out = pl.pallas_call(
        _body,
        grid_spec=grid_spec,
        out_shape=jax.ShapeDtypeStruct((B, 1, 3 * D), jnp.float32),
    )(scal, atom_feats, bond_feats, global_feats.reshape(B, 1, D),
      w_atom.reshape(1, D), w_bond.reshape(1, D))
    return out.reshape(B, 3 * D)


# speedup vs baseline: 21.0775x; 3.0107x over previous
import jax
import jax.numpy as jnp
from jax.experimental import pallas as pl
from jax.experimental.pallas import tpu as pltpu

_WSEG = 8
_RBLK = 2048


def _body(scal, af_hbm, bf_hbm, lo_a, hi_a, lo_b, hi_b, gf_blk, wa, wb,
          out_blk, va, vb, sems):
    w = pl.program_id(0)
    nwin = pl.num_programs(0)
    slot = jax.lax.rem(w, 2)

    def issue(win, slot):
        pltpu.make_async_copy(
            af_hbm.at[pl.ds(scal[0, win], _RBLK)], va.at[slot], sems.at[0, slot]
        ).start()
        pltpu.make_async_copy(
            bf_hbm.at[pl.ds(scal[1, win], _RBLK)], vb.at[slot], sems.at[1, slot]
        ).start()

    @pl.when(w == 0)
    def _():
        issue(0, 0)

    @pl.when(w + 1 < nwin)
    def _():
        issue(w + 1, 1 - slot)

    pltpu.make_async_copy(
        af_hbm.at[pl.ds(scal[0, w], _RBLK)], va.at[slot], sems.at[0, slot]
    ).wait()
    pltpu.make_async_copy(
        bf_hbm.at[pl.ds(scal[1, w], _RBLK)], vb.at[slot], sems.at[1, slot]
    ).wait()

    riota = jax.lax.broadcasted_iota(jnp.int32, (_RBLK, _WSEG), 0)

    def branch(x, lo, hi, w_row):
        score = jnp.sum(x * w_row, axis=1, keepdims=True)
        score = jnp.where(score >= 0.0, score, 0.2 * score)
        mask = (riota >= lo[0]) & (riota < hi[0])
        m = jnp.max(jnp.where(mask, score, -jnp.inf), axis=0, keepdims=True)
        m = jnp.where(jnp.isfinite(m), m, 0.0)
        e = jnp.where(mask, jnp.exp(score - m), 0.0)
        s = jnp.sum(e, axis=0, keepdims=True)
        alpha = e / jnp.where(s > 0.0, s, 1.0)
        return jax.lax.dot_general(
            alpha, x, (((0,), (0,)), ((), ())),
            preferred_element_type=jnp.float32)

    ra = branch(va[slot], lo_a, hi_a, wa[...])
    rb = branch(vb[slot], lo_b, hi_b, wb[...])
    out_blk[0, :, 0:128] = ra
    out_blk[0, :, 128:256] = rb
    out_blk[0, :, 256:384] = gf_blk[0]


def kernel(atom_feats, atom_sizes, bond_feats, bond_sizes, global_feats, w_atom, w_bond):
    N, D = atom_feats.shape
    B = global_feats.shape[0]
    nwin = B // _WSEG

    def mk(sizes):
        sizes = sizes.astype(jnp.int32)
        cs = jnp.cumsum(sizes)
        starts = jnp.concatenate([jnp.zeros((1,), jnp.int32), cs[:-1]])
        base = starts[:: _WSEG]
        base_c = jnp.minimum(base, N - _RBLK)
        lo = starts.reshape(nwin, _WSEG) - base_c[:, None]
        hi = lo + sizes.reshape(nwin, _WSEG)
        return base_c, lo.reshape(nwin, 1, _WSEG), hi.reshape(nwin, 1, _WSEG)

    ba, lo_a, hi_a = mk(atom_sizes)
    bb, lo_b, hi_b = mk(bond_sizes)
    scal = jnp.stack([ba, bb])

    win_spec = pl.BlockSpec((1, 1, _WSEG), lambda w, s: (w, 0, 0))
    grid_spec = pltpu.PrefetchScalarGridSpec(
        num_scalar_prefetch=1,
        grid=(nwin,),
        in_specs=[
            pl.BlockSpec(memory_space=pltpu.MemorySpace.HBM),
            pl.BlockSpec(memory_space=pltpu.MemorySpace.HBM),
            win_spec, win_spec,
            win_spec, win_spec,
            pl.BlockSpec((1, _WSEG, D), lambda w, s: (w, 0, 0)),
            pl.BlockSpec((1, D), lambda w, s: (0, 0)),
            pl.BlockSpec((1, D), lambda w, s: (0, 0)),
        ],
        out_specs=pl.BlockSpec((1, _WSEG, 3 * D), lambda w, s: (w, 0, 0)),
        scratch_shapes=[
            pltpu.VMEM((2, _RBLK, D), jnp.float32),
            pltpu.VMEM((2, _RBLK, D), jnp.float32),
            pltpu.SemaphoreType.DMA((2, 2)),
        ],
    )
    out = pl.pallas_call(
        _body,
        grid_spec=grid_spec,
        out_shape=jax.ShapeDtypeStruct((nwin, _WSEG, 3 * D), jnp.float32),
    )(scal, atom_feats, bond_feats, lo_a, hi_a, lo_b, hi_b,
      global_feats.reshape(nwin, _WSEG, D),
      w_atom.reshape(1, D), w_bond.reshape(1, D))
    return out.reshape(B, 3 * D)
